# baseline (speedup 1.0000x reference)
import jax
import jax.numpy as jnp
from jax.experimental import pallas as pl

_N_ATOMS = 100000
_N_SPLINE = 10000
_CUTOFF = 0.6
_BOX = 1.0


def _cubic(c0, c1, c2, c3, dx):
    return c3 + dx * (c2 + dx * (c1 + dx * c0))


def kernel(coords, edge_index, atom_types, spline_r_x, density_coeffs, embed_x, embed_coeffs, pair_coeffs):
    row = edge_index[0]
    col = edge_index[1]
    dvec = coords[row] - coords[col]
    dvec = dvec - jnp.round(dvec / _BOX) * _BOX
    dist = jnp.sqrt(jnp.sum(dvec * dvec, axis=1) + 1e-12)
    in_cut = dist < _CUTOFF
    r = jnp.clip(dist, spline_r_x[0], spline_r_x[-1])
    idx = jnp.clip(jnp.searchsorted(spline_r_x, r, side='right') - 1, 0, _N_SPLINE - 2)
    dx = r - spline_r_x[idx]
    ti = atom_types[row]
    tj = atom_types[col]
    d0 = density_coeffs[tj, 0, idx]
    d1 = density_coeffs[tj, 1, idx]
    d2 = density_coeffs[tj, 2, idx]
    d3 = density_coeffs[tj, 3, idx]
    dens = jnp.where(in_cut, _cubic(d0, d1, d2, d3, dx), 0.0)
    rho = jax.ops.segment_sum(dens, row, num_segments=_N_ATOMS)
    F = jnp.zeros((_N_ATOMS,), jnp.float32)
    for t in range(2):
        gx = embed_x[t]
        rc = jnp.clip(rho, gx[0], gx[-1])
        ei = jnp.clip(jnp.searchsorted(gx, rc, side='right') - 1, 0, _N_SPLINE - 2)
        edx = rc - gx[ei]
        ev = _cubic(embed_coeffs[t, 0, ei], embed_coeffs[t, 1, ei], embed_coeffs[t, 2, ei], embed_coeffs[t, 3, ei], edx)
        F = jnp.where(atom_types == t, ev, F)
    p0 = pair_coeffs[ti, tj, 0, idx]
    p1 = pair_coeffs[ti, tj, 1, idx]
    p2 = pair_coeffs[ti, tj, 2, idx]
    p3 = pair_coeffs[ti, tj, 3, idx]
    pv = jnp.where(in_cut, _cubic(p0, p1, p2, p3, dx), 0.0)
    total_energy = jnp.sum(F) + 0.5 * jnp.sum(pv)
    return total_energy


# speedup vs baseline: 506.4086x; 506.4086x over previous
import jax
import jax.numpy as jnp
from jax import lax
from jax.experimental import pallas as pl
from jax.experimental.pallas import tpu as pltpu
from jax.experimental.pallas import tpu_sc as plsc

_N_ATOMS = 100000
_N_EDGES = 6400000
_N_SPLINE = 10000
_NW = 32
_CHUNK_ROWS = 4
_N_ROWS = _N_EDGES // 128
_N_CHUNKS = _N_ROWS // _CHUNK_ROWS
_ITERS = (_N_CHUNKS + _NW - 1) // _NW
_A_CHUNKS = 782
_A_ITERS = (_A_CHUNKS + _NW - 1) // _NW
_NCOEF = 4 * (_N_SPLINE - 1)


def _iota16():
    return lax.iota(jnp.int32, 16)


def _splat(x, dtype=jnp.float32):
    return jnp.full((16,), x, dtype)


def _newton_sqrt(u):
    i = lax.bitcast_convert_type(u, jnp.int32)
    i = jnp.int32(0x5F3759DF) - lax.shift_right_logical(i, 1)
    y = lax.bitcast_convert_type(i, jnp.float32)
    half_u = jnp.float32(0.5) * u
    y = y * (jnp.float32(1.5) - half_u * y * y)
    y = y * (jnp.float32(1.5) - half_u * y * y)
    s = u * y
    return jnp.float32(0.5) * (s + u / s)


def _square_exact(a):
    p = a * a
    sp = a * jnp.float32(4097.0)
    ah = sp - (sp - a)
    al = a - ah
    e = ((ah * ah - p) + jnp.float32(2.0) * ah * al) + al * al
    return p, e


def _two_sum(a, b):
    s = a + b
    bb = s - a
    err = (a - (s - bb)) + (b - bb)
    return s, err


def _exact_sumsq_eps(x, y, z):
    p1, e1 = _square_exact(x)
    p2, e2 = _square_exact(y)
    p3, e3 = _square_exact(z)
    s12, t12 = _two_sum(p1, p2)
    s, t3 = _two_sum(s12, p3)
    err = ((((t12 + t3) + e1) + e2) + e3) + jnp.float32(1e-12)
    return s + err


def _wrap(d):
    return jnp.where(d > jnp.float32(0.5), d - jnp.float32(1.0),
                     jnp.where(d < jnp.float32(-0.5), d + jnp.float32(1.0), d))


def _edge_kernel(row2, col2, atom_tbl, coeff_tbl, sbin, ucut16, invh16,
                 rho_init, rho_out, pv_out,
                 atom_sp, coeff_sp, rho_sp,
                 sbin_v, ucut_v, invh_v,
                 rowv, colv, arow, acol, crows, cidx, dxb, mfb, densb, pvb,
                 sem_in, sem_g, sem_s):
    c = lax.axis_index("c")
    s = lax.axis_index("s")
    w = s * 2 + c

    pltpu.sync_copy(sbin, sbin_v)
    pltpu.sync_copy(ucut16, ucut_v)
    pltpu.sync_copy(invh16, invh_v)

    @pl.when(s == 0)
    def _stage():
        pltpu.sync_copy(atom_tbl, atom_sp)
        pltpu.sync_copy(coeff_tbl, coeff_sp)
        pltpu.sync_copy(rho_init, rho_sp)

    plsc.subcore_barrier()

    ucut = ucut_v[...]
    invh = invh_v[...]

    def chunk_body(it, pvacc):
        cc = w + it * _NW
        valid = cc < _N_CHUNKS
        cce = jnp.minimum(cc, _N_CHUNKS - 1)
        r0 = cce * _CHUNK_ROWS

        pltpu.async_copy(row2.at[pl.ds(r0, _CHUNK_ROWS)], rowv, sem_in).wait()
        pltpu.async_copy(col2.at[pl.ds(r0, _CHUNK_ROWS)], colv, sem_in).wait()

        hs = []
        for k in range(_CHUNK_ROWS):
            hs.append(pltpu.async_copy(
                atom_sp.at[rowv.at[k]], arow.at[pl.ds(k * 128, 128)], sem_g))
            hs.append(pltpu.async_copy(
                atom_sp.at[colv.at[k]], acol.at[pl.ds(k * 128, 128)], sem_g))
        for h in hs:
            h.wait()

        validf = jnp.where(valid, jnp.float32(1.0), jnp.float32(0.0))
        validv = _splat(validf)

        lanes = _iota16()
        one = jnp.full((16,), 1, jnp.int32)
        zero = jnp.full((16,), 0, jnp.int32)

        for jj in range(_CHUNK_ROWS * 8):
            sl = pl.ds(jj * 16, 16)
            li = jj * 16 + lanes

            def lg(ref, comp):
                return plsc.load_gather(ref, [li, jnp.full((16,), comp, jnp.int32)])

            xr = lg(arow, 0); yr = lg(arow, 1); zr = lg(arow, 2); tr = lg(arow, 3)
            xc = lg(acol, 0); yc = lg(acol, 1); zc = lg(acol, 2); tc = lg(acol, 3)
            wx = _wrap(xr - xc)
            wy = _wrap(yr - yc)
            wz = _wrap(zr - zc)
            u = _exact_sumsq_eps(wx, wy, wz)
            mf = jnp.where(u < ucut, validv, _splat(0.0))
            dist = _newton_sqrt(u)
            ig = lax.convert_element_type(dist * invh, jnp.int32)
            ig = jnp.clip(ig, 0, _N_SPLINE - 2)
            ig2 = ig + ig
            t0 = plsc.load_gather(sbin_v, [ig2])
            t1 = plsc.load_gather(sbin_v, [ig2 + 2])
            idx = ig - jnp.where(u < t0, one, zero) + jnp.where(u >= t1, one, zero)
            idx = jnp.clip(idx, 0, _N_SPLINE - 2)
            knot = plsc.load_gather(sbin_v, [idx + idx + 1])
            dx = dist - knot
            ti = jnp.clip(lax.convert_element_type(tr, jnp.int32), 0, 1)
            tj = jnp.clip(lax.convert_element_type(tc, jnp.int32), 0, 1)
            ci = (ti + ti + tj) * (_N_SPLINE - 1) + idx

            cidx[jj * 16 // 128, pl.ds((jj * 16) % 128, 16)] = ci
            dxb[sl] = dx
            mfb[sl] = mf

        hs2 = []
        for k in range(_CHUNK_ROWS):
            hs2.append(pltpu.async_copy(
                coeff_sp.at[cidx.at[k]], crows.at[pl.ds(k * 128, 128)], sem_g))
        for h in hs2:
            h.wait()

        for jj in range(_CHUNK_ROWS * 8):
            sl = pl.ds(jj * 16, 16)
            li = jj * 16 + lanes

            def cg(comp):
                return plsc.load_gather(crows, [li, jnp.full((16,), comp, jnp.int32)])

            p0 = cg(0); p1 = cg(1); p2 = cg(2); p3 = cg(3)
            d0 = cg(4); d1 = cg(5); d2 = cg(6); d3 = cg(7)
            dx = dxb[sl]
            mf = mfb[sl]
            dens = mf * (d3 + dx * (d2 + dx * (d1 + dx * d0)))
            pvv = mf * (p3 + dx * (p2 + dx * (p1 + dx * p0)))
            densb[sl] = dens
            pvacc = pvacc + pvv

        for k in range(_CHUNK_ROWS):
            pltpu.sync_copy(densb.at[pl.ds(k * 128, 128)],
                            rho_sp.at[rowv.at[k]], add=True)
        return pvacc

    pvacc = lax.fori_loop(0, _ITERS, chunk_body, jnp.zeros((16,), jnp.float32))

    pvb[...] = pvacc
    pltpu.sync_copy(pvb, pv_out.at[w])

    plsc.subcore_barrier()

    @pl.when(s == 0)
    def _flush():
        pltpu.sync_copy(rho_sp, rho_out.at[c])


def _atom_kernel(rho2, types_p, egx, ecoef, invhe16,
                 f_out,
                 egx_v, ecoef_v, invhe_v, r0v, r1v, tv, fb,
                 sem_in):
    c = lax.axis_index("c")
    s = lax.axis_index("s")
    w = s * 2 + c

    pltpu.sync_copy(egx, egx_v)
    pltpu.sync_copy(ecoef, ecoef_v)
    pltpu.sync_copy(invhe16, invhe_v)

    lanes = _iota16()
    invhe = invhe_v[...]

    def it_body(it, facc):
        cc = w + it * _NW
        valid = cc < _A_CHUNKS
        cce = jnp.minimum(cc, _A_CHUNKS - 1)
        a0 = cce * 128
        pltpu.async_copy(rho2.at[0].at[pl.ds(a0, 128)], r0v, sem_in).wait()
        pltpu.async_copy(rho2.at[1].at[pl.ds(a0, 128)], r1v, sem_in).wait()
        pltpu.async_copy(types_p.at[pl.ds(a0, 128)], tv, sem_in).wait()
        validf = jnp.where(valid, jnp.float32(1.0), jnp.float32(0.0))

        acc = facc
        for jj in range(8):
            off = jj * 16
            rho = r0v[pl.ds(off, 16)] + r1v[pl.ds(off, 16)]
            t = tv[pl.ds(off, 16)]
            tb = t * _N_SPLINE
            gxlo = plsc.load_gather(egx_v, [tb])
            gxhi = plsc.load_gather(egx_v, [tb + (_N_SPLINE - 1)])
            rc = jnp.minimum(jnp.maximum(rho, gxlo), gxhi)
            ig = lax.convert_element_type((rc - gxlo) * invhe, jnp.int32)
            ig = jnp.clip(ig, 0, _N_SPLINE - 2)
            e0 = plsc.load_gather(egx_v, [tb + ig])
            e1 = plsc.load_gather(egx_v, [tb + ig + 1])
            one = jnp.full((16,), 1, jnp.int32)
            zero = jnp.full((16,), 0, jnp.int32)
            ei = ig - jnp.where(rc < e0, one, zero) + jnp.where(rc >= e1, one, zero)
            ei = jnp.clip(ei, 0, _N_SPLINE - 2)
            ek = plsc.load_gather(egx_v, [tb + ei])
            edx = rc - ek
            cb = (t * (_N_SPLINE - 1) + ei) * 4

            def eg(comp):
                return plsc.load_gather(ecoef_v, [cb + comp])

            c0 = eg(0); c1 = eg(1); c2 = eg(2); c3 = eg(3)
            fv = c3 + edx * (c2 + edx * (c1 + edx * c0))
            aid = jnp.full((16,), a0 + off, jnp.int32) + lanes
            mval = jnp.where(aid < _N_ATOMS, _splat(validf), _splat(0.0))
            acc = acc + fv * mval
        return acc

    facc = lax.fori_loop(0, _A_ITERS, it_body, jnp.zeros((16,), jnp.float32))
    fb[...] = facc
    pltpu.sync_copy(fb, f_out.at[w])


def kernel(coords, edge_index, atom_types, spline_r_x, density_coeffs, embed_x, embed_coeffs, pair_coeffs):
    row2 = edge_index[0].reshape(_N_ROWS, 128)
    col2 = edge_index[1].reshape(_N_ROWS, 128)
    atom_tbl = jnp.concatenate(
        [coords, atom_types.astype(jnp.float32)[:, None],
         jnp.zeros((_N_ATOMS, 4), jnp.float32)], axis=1)

    knots = spline_r_x
    c0 = knots * knots
    c0i = lax.bitcast_convert_type(c0, jnp.int32)
    cands = jnp.stack([lax.bitcast_convert_type(jnp.maximum(c0i + j, 0), jnp.float32)
                       for j in range(-6, 7)], axis=0)
    ok = jnp.sqrt(cands) >= knots[None, :]
    big = jnp.float32(3.4e38)
    uthr = jnp.min(jnp.where(ok, cands, big), axis=0)
    uthr = uthr.at[0].set(jnp.float32(0.0))
    ucut = uthr[_N_SPLINE - 1]
    sbin = jnp.stack([uthr, knots], axis=1).reshape(-1)
    inv_h = jnp.float32(_N_SPLINE - 1) / (knots[-1] - knots[0])
    ucut16 = jnp.full((16,), ucut, jnp.float32)
    invh16 = jnp.full((16,), inv_h, jnp.float32)

    pair_t = jnp.transpose(pair_coeffs, (0, 1, 3, 2))
    dens_t = jnp.transpose(density_coeffs, (0, 2, 1))
    dens_b = jnp.broadcast_to(dens_t[None], (2, 2, _N_SPLINE - 1, 4))
    coeff_tbl = jnp.concatenate([pair_t, dens_b], axis=3).reshape(-1, 8)

    rho_init = jnp.zeros((_N_ATOMS,), jnp.float32)

    rho_part, pv_part = pl.kernel(
        _edge_kernel,
        out_type=[
            jax.ShapeDtypeStruct((2, _N_ATOMS), jnp.float32),
            jax.ShapeDtypeStruct((_NW, 16), jnp.float32),
        ],
        mesh=plsc.VectorSubcoreMesh(core_axis_name="c", subcore_axis_name="s"),
        compiler_params=pltpu.CompilerParams(needs_layout_passes=False, use_tc_tiling_on_sc=False),
        scratch_types=[
            pltpu.VMEM_SHARED((_N_ATOMS, 8), jnp.float32),
            pltpu.VMEM_SHARED((_NCOEF, 8), jnp.float32),
            pltpu.VMEM_SHARED((_N_ATOMS,), jnp.float32),
            pltpu.VMEM((2 * _N_SPLINE,), jnp.float32),
            pltpu.VMEM((16,), jnp.float32),
            pltpu.VMEM((16,), jnp.float32),
            pltpu.VMEM((_CHUNK_ROWS, 128), jnp.int32),
            pltpu.VMEM((_CHUNK_ROWS, 128), jnp.int32),
            pltpu.VMEM((_CHUNK_ROWS * 128, 8), jnp.float32),
            pltpu.VMEM((_CHUNK_ROWS * 128, 8), jnp.float32),
            pltpu.VMEM((_CHUNK_ROWS * 128, 8), jnp.float32),
            pltpu.VMEM((_CHUNK_ROWS, 128), jnp.int32),
            pltpu.VMEM((_CHUNK_ROWS * 128,), jnp.float32),
            pltpu.VMEM((_CHUNK_ROWS * 128,), jnp.float32),
            pltpu.VMEM((_CHUNK_ROWS * 128,), jnp.float32),
            pltpu.VMEM((16,), jnp.float32),
            pltpu.SemaphoreType.DMA,
            pltpu.SemaphoreType.DMA,
            pltpu.SemaphoreType.DMA,
        ],
    )(row2, col2, atom_tbl, coeff_tbl, sbin, ucut16, invh16, rho_init)

    rho2 = jnp.concatenate(
        [rho_part, jnp.zeros((2, _A_CHUNKS * 128 - _N_ATOMS), jnp.float32)], axis=1)
    types_p = jnp.concatenate(
        [atom_types, jnp.zeros((_A_CHUNKS * 128 - _N_ATOMS,), jnp.int32)])
    egx = embed_x.reshape(-1)
    ecoef = jnp.transpose(embed_coeffs, (0, 2, 1)).reshape(-1)
    inv_he = jnp.float32(_N_SPLINE - 1) / (embed_x[0, -1] - embed_x[0, 0])
    invhe16 = jnp.full((16,), inv_he, jnp.float32)

    f_part = pl.kernel(
        _atom_kernel,
        out_type=jax.ShapeDtypeStruct((_NW, 16), jnp.float32),
        mesh=plsc.VectorSubcoreMesh(core_axis_name="c", subcore_axis_name="s"),
        compiler_params=pltpu.CompilerParams(needs_layout_passes=False, use_tc_tiling_on_sc=False),
        scratch_types=[
            pltpu.VMEM((2 * _N_SPLINE,), jnp.float32),
            pltpu.VMEM((4 * 2 * (_N_SPLINE - 1),), jnp.float32),
            pltpu.VMEM((16,), jnp.float32),
            pltpu.VMEM((128,), jnp.float32),
            pltpu.VMEM((128,), jnp.float32),
            pltpu.VMEM((128,), jnp.int32),
            pltpu.VMEM((16,), jnp.float32),
            pltpu.SemaphoreType.DMA,
        ],
    )(rho2, types_p, egx, ecoef, invhe16)

    return jnp.sum(f_part) + jnp.float32(0.5) * jnp.sum(pv_part)
